# Initial kernel scaffold; baseline (speedup 1.0000x reference)
import jax
import jax.numpy as jnp
from jax.experimental import pallas as pl


def kernel(doy, pe):
    raise NotImplementedError("write your pallas kernel here")


# speedup vs baseline: 6.9244x; 6.9244x over previous
import functools

import jax
import jax.numpy as jnp
from jax import lax
from jax.experimental import pallas as pl
from jax.experimental.pallas import tpu as pltpu
from jax.experimental.pallas import tpu_sc as plsc

B, S, D = 4096, 200, 128
TOT = B * S
NC, NS = 2, 16
NW = NC * NS
PER_W = TOT // NW
CH = 128
NCH = PER_W // CH

_mesh = plsc.VectorSubcoreMesh(core_axis_name="c", subcore_axis_name="s")


@functools.partial(
    pl.kernel,
    out_type=jax.ShapeDtypeStruct((TOT, D), jnp.float32),
    mesh=_mesh,
    scratch_types=[
        pltpu.VMEM((PER_W,), jnp.int32),
        pltpu.VMEM((CH, D), jnp.float32),
        pltpu.SemaphoreType.DMA,
    ],
)
def _pe_gather(doy_hbm, pe_hbm, out_hbm, idx_v, rows_v, sem):
    wid = lax.axis_index("s") * NC + lax.axis_index("c")
    base = wid * PER_W
    pltpu.sync_copy(doy_hbm.at[pl.ds(base, PER_W)], idx_v)

    def step(g, carry):
        off = g * CH
        pltpu.async_copy(
            pe_hbm.at[idx_v.at[pl.ds(off, CH)]], rows_v, sem
        ).wait()
        pltpu.sync_copy(rows_v, out_hbm.at[pl.ds(base + off, CH)])
        return carry

    lax.fori_loop(0, NCH, step, 0)


def kernel(doy, pe):
    flat = _pe_gather(doy.reshape(TOT).astype(jnp.int32), pe)
    return flat.reshape(B, S, D)


# speedup vs baseline: 9.8504x; 1.4226x over previous
import functools

import jax
import jax.numpy as jnp
from jax import lax
from jax.experimental import pallas as pl
from jax.experimental.pallas import tpu as pltpu
from jax.experimental.pallas import tpu_sc as plsc

B, S, D = 4096, 200, 128
TOT = B * S
NC, NS = 2, 16
NW = NC * NS
PER_W = TOT // NW
CH = 128
SUP = 256
KG = SUP // CH
NSUP = PER_W // SUP
NHALF = NSUP // 2

_mesh = plsc.VectorSubcoreMesh(core_axis_name="c", subcore_axis_name="s")


@functools.partial(
    pl.kernel,
    out_type=jax.ShapeDtypeStruct((TOT, D), jnp.float32),
    mesh=_mesh,
    scratch_types=[
        pltpu.VMEM((PER_W,), jnp.int32),
        pltpu.VMEM((SUP, D), jnp.float32),
        pltpu.VMEM((SUP, D), jnp.float32),
        pltpu.SemaphoreType.DMA,
        pltpu.SemaphoreType.DMA,
        pltpu.SemaphoreType.DMA,
        pltpu.SemaphoreType.DMA,
    ],
)
def _pe_gather(doy_hbm, pe_hbm, out_hbm, idx_v, rows_a, rows_b,
               gsem_a, gsem_b, wsem_a, wsem_b):
    wid = lax.axis_index("s") * NC + lax.axis_index("c")
    base = wid * PER_W
    pltpu.sync_copy(doy_hbm.at[pl.ds(base, PER_W)], idx_v)

    def fire(rows, gsem, s):
        off = s * SUP
        for j in range(KG):
            pltpu.async_copy(
                pe_hbm.at[idx_v.at[pl.ds(off + j * CH, CH)]],
                rows.at[pl.ds(j * CH, CH)], gsem)

    def drain(rows, gsem):
        pltpu.make_async_copy(out_hbm.at[pl.ds(0, SUP)], rows, gsem).wait()

    fire(rows_a, gsem_a, 0)

    def pair(i, carry):
        s_a = 2 * i
        s_b = s_a + 1
        fire(rows_b, gsem_b, s_b)
        drain(rows_a, gsem_a)
        wa = pltpu.async_copy(rows_a, out_hbm.at[pl.ds(base + s_a * SUP, SUP)],
                              wsem_a)
        wa.wait()

        @pl.when(i < NHALF - 1)
        def _():
            fire(rows_a, gsem_a, s_a + 2)

        drain(rows_b, gsem_b)
        wb = pltpu.async_copy(rows_b, out_hbm.at[pl.ds(base + s_b * SUP, SUP)],
                              wsem_b)
        wb.wait()
        return carry

    lax.fori_loop(0, NHALF, pair, 0)


def kernel(doy, pe):
    flat = _pe_gather(doy.reshape(TOT).astype(jnp.int32), pe)
    return flat.reshape(B, S, D)


# speedup vs baseline: 9.8599x; 1.0010x over previous
import functools

import jax
import jax.numpy as jnp
from jax import lax
from jax.experimental import pallas as pl
from jax.experimental.pallas import tpu as pltpu
from jax.experimental.pallas import tpu_sc as plsc

B, S, D = 4096, 200, 128
TOT = B * S
NC, NS = 2, 16
NW = NC * NS
PER_W = TOT // NW
SUP = 128
NSUP = PER_W // SUP
R = 5
G = 3
W = 2
NSTEP = NSUP // R

_mesh = plsc.VectorSubcoreMesh(core_axis_name="c", subcore_axis_name="s")


@functools.partial(
    pl.kernel,
    out_type=jax.ShapeDtypeStruct((TOT, D), jnp.float32),
    mesh=_mesh,
    scratch_types=[
        pltpu.VMEM((PER_W,), jnp.int32),
    ] + [pltpu.VMEM((SUP, D), jnp.float32) for _ in range(R)]
      + [pltpu.SemaphoreType.DMA for _ in range(2 * R)],
)
def _pe_gather(doy_hbm, pe_hbm, out_hbm, idx_v,
               r0, r1, r2, r3, r4,
               g0, g1, g2, g3, g4, w0, w1, w2, w3, w4):
    rows = (r0, r1, r2, r3, r4)
    gsem = (g0, g1, g2, g3, g4)
    wsem = (w0, w1, w2, w3, w4)
    wid = lax.axis_index("s") * NC + lax.axis_index("c")
    base = wid * PER_W
    pltpu.sync_copy(doy_hbm.at[pl.ds(base, PER_W)], idx_v)

    def fire(b, s):
        pltpu.async_copy(
            pe_hbm.at[idx_v.at[pl.ds(s * SUP, SUP)]], rows[b], gsem[b])

    for b in range(G):
        fire(b, b)

    def step(i, carry):
        s0 = R * i
        for b in range(R):
            s = s0 + b
            pltpu.make_async_copy(out_hbm.at[pl.ds(0, SUP)], rows[b],
                                  gsem[b]).wait()
            pltpu.async_copy(rows[b], out_hbm.at[pl.ds(base + s * SUP, SUP)],
                             wsem[b])
            bn = (b + G) % R

            @pl.when(s >= W)
            def _():
                pltpu.make_async_copy(rows[bn], out_hbm.at[pl.ds(0, SUP)],
                                      wsem[bn]).wait()

            @pl.when(s + G < NSUP)
            def _():
                fire(bn, s + G)
        return carry

    lax.fori_loop(0, NSTEP, step, 0)
    for k in range(W):
        b = (NSUP - W + k) % R
        pltpu.make_async_copy(rows[b], out_hbm.at[pl.ds(0, SUP)],
                              wsem[b]).wait()


def kernel(doy, pe):
    flat = _pe_gather(doy.reshape(TOT).astype(jnp.int32), pe)
    return flat.reshape(B, S, D)


# speedup vs baseline: 17.2529x; 1.7498x over previous
import functools

import jax
import jax.numpy as jnp
from jax import lax
from jax.experimental import pallas as pl
from jax.experimental.pallas import tpu as pltpu
from jax.experimental.pallas import tpu_sc as plsc

B, S, D = 4096, 200, 128
V = 5000
TOT = B * S
NC, NS = 2, 16
NW = NC * NS
PER_W = TOT // NW
SUP = 128
NSUP = PER_W // SUP
R = 3
G = 2
W = 1
NSTEP = (NSUP - 2) // R

_mesh = plsc.VectorSubcoreMesh(core_axis_name="c", subcore_axis_name="s")


@functools.partial(
    pl.kernel,
    out_type=jax.ShapeDtypeStruct((TOT, D), jnp.float32),
    mesh=_mesh,
    scratch_types=[
        pltpu.VMEM((PER_W,), jnp.int32),
        pltpu.VMEM_SHARED((V, D), jnp.float32),
    ] + [pltpu.VMEM((SUP, D), jnp.float32) for _ in range(R)]
      + [pltpu.SemaphoreType.DMA for _ in range(2 * R)],
)
def _pe_gather(doy_hbm, pe_hbm, out_hbm, idx_v, pe_sp,
               r0, r1, r2, g0, g1, g2, w0, w1, w2):
    rows = (r0, r1, r2)
    gsem = (g0, g1, g2)
    wsem = (w0, w1, w2)
    sid = lax.axis_index("s")
    wid = sid * NC + lax.axis_index("c")
    base = wid * PER_W

    @pl.when(sid < 8)
    def _():
        pltpu.sync_copy(pe_hbm.at[pl.ds(sid * 624, 624)],
                        pe_sp.at[pl.ds(sid * 624, 624)])

    @pl.when(sid == 8)
    def _():
        pltpu.sync_copy(pe_hbm.at[pl.ds(4992, 8)], pe_sp.at[pl.ds(4992, 8)])

    pltpu.sync_copy(doy_hbm.at[pl.ds(base, PER_W)], idx_v)
    plsc.subcore_barrier()

    def fire(b, s):
        pltpu.async_copy(
            pe_sp.at[idx_v.at[pl.ds(s * SUP, SUP)]], rows[b], gsem[b])

    def drain_gather(b):
        pltpu.make_async_copy(out_hbm.at[pl.ds(0, SUP)], rows[b],
                              gsem[b]).wait()

    def drain_write(b):
        pltpu.make_async_copy(rows[b], out_hbm.at[pl.ds(0, SUP)],
                              wsem[b]).wait()

    for b in range(G):
        fire(b, b)

    def step(i, carry):
        s0 = R * i
        for b in range(R):
            s = s0 + b
            drain_gather(b)
            pltpu.async_copy(rows[b], out_hbm.at[pl.ds(base + s * SUP, SUP)],
                             wsem[b])
            bn = (b + G) % R

            @pl.when(s >= W)
            def _():
                drain_write(bn)

            fire(bn, s + G)
        return carry

    lax.fori_loop(0, NSTEP, step, 0)

    for s in (NSUP - 2, NSUP - 1):
        b = s % R
        drain_gather(b)
        pltpu.async_copy(rows[b], out_hbm.at[pl.ds(base + s * SUP, SUP)],
                         wsem[b])
        drain_write((b + G) % R)
    drain_write((NSUP - 1) % R)


def kernel(doy, pe):
    flat = _pe_gather(doy.reshape(TOT).astype(jnp.int32), pe)
    return flat.reshape(B, S, D)


# speedup vs baseline: 17.8901x; 1.0369x over previous
import functools

import jax
import jax.numpy as jnp
from jax import lax
from jax.experimental import pallas as pl
from jax.experimental.pallas import tpu as pltpu
from jax.experimental.pallas import tpu_sc as plsc

B, S, D = 4096, 200, 128
V = 5000
TOT = B * S
NC, NS = 2, 16
NW = NC * NS
PER_W = TOT // NW
SUP = 64
NSUP = PER_W // SUP
R = 5
G = 3
W = 2
NSTEP = NSUP // R

_mesh = plsc.VectorSubcoreMesh(core_axis_name="c", subcore_axis_name="s")


@functools.partial(
    pl.kernel,
    out_type=jax.ShapeDtypeStruct((TOT, D), jnp.float32),
    mesh=_mesh,
    scratch_types=[
        pltpu.VMEM((PER_W,), jnp.int32),
        pltpu.VMEM_SHARED((V, D), jnp.float32),
    ] + [pltpu.VMEM((SUP, D), jnp.float32) for _ in range(R)]
      + [pltpu.SemaphoreType.DMA for _ in range(2 * R)],
)
def _pe_gather(doy_hbm, pe_hbm, out_hbm, idx_v, pe_sp,
               r0, r1, r2, r3, r4,
               g0, g1, g2, g3, g4, w0, w1, w2, w3, w4):
    rows = (r0, r1, r2, r3, r4)
    gsem = (g0, g1, g2, g3, g4)
    wsem = (w0, w1, w2, w3, w4)
    sid = lax.axis_index("s")
    wid = sid * NC + lax.axis_index("c")
    base = wid * PER_W

    @pl.when(sid < 8)
    def _():
        pltpu.sync_copy(pe_hbm.at[pl.ds(sid * 624, 624)],
                        pe_sp.at[pl.ds(sid * 624, 624)])

    @pl.when(sid == 8)
    def _():
        pltpu.sync_copy(pe_hbm.at[pl.ds(4992, 8)], pe_sp.at[pl.ds(4992, 8)])

    pltpu.sync_copy(doy_hbm.at[pl.ds(base, PER_W)], idx_v)
    plsc.subcore_barrier()

    def fire(b, s):
        pltpu.async_copy(
            pe_sp.at[idx_v.at[pl.ds(s * SUP, SUP)]], rows[b], gsem[b])

    def drain_gather(b):
        pltpu.make_async_copy(out_hbm.at[pl.ds(0, SUP)], rows[b],
                              gsem[b]).wait()

    def drain_write(b):
        pltpu.make_async_copy(rows[b], out_hbm.at[pl.ds(0, SUP)],
                              wsem[b]).wait()

    for b in range(G):
        fire(b, b)

    def step(i, carry):
        s0 = R * i
        for b in range(R):
            s = s0 + b
            drain_gather(b)
            pltpu.async_copy(rows[b], out_hbm.at[pl.ds(base + s * SUP, SUP)],
                             wsem[b])
            bn = (b + G) % R

            @pl.when(s >= W)
            def _():
                drain_write(bn)

            @pl.when(s + G < NSUP)
            def _():
                fire(bn, s + G)
        return carry

    lax.fori_loop(0, NSTEP, step, 0)

    for k in range(W):
        drain_write((NSUP - W + k) % R)


def kernel(doy, pe):
    flat = _pe_gather(doy.reshape(TOT).astype(jnp.int32), pe)
    return flat.reshape(B, S, D)
